# baseline (speedup 1.0000x reference)
import functools

import jax
import jax.numpy as jnp
from jax import lax
from jax.experimental import pallas as pl
from jax.experimental.pallas import tpu as pltpu
from jax.experimental.pallas import tpu_sc as plsc

_B = 2048
_D = 2048
_E = 8
_NC = 16
_FB = 16
_FPE = 2
_H = 1024
_RH = 1024
_K = 2
_EPAD = 128

_TMR = 256
_TM = 128
_KP = _B * _K
_NT = _KP // _TM
_NW = _NT + _E - 1
_NWORK = 32


def _gelu(x):
    return 0.5 * x * (1.0 + lax.erf(x * 0.7071067811865476))


def _router_body(hid, st, g, b, w1h, w1f, b1, w2, b2, h_out, idx_out, wp_out):
    x = hid[...]
    mu = jnp.mean(x, axis=-1, keepdims=True)
    xc = x - mu
    var = jnp.mean(xc * xc, axis=-1, keepdims=True)
    hn = xc * lax.rsqrt(var + 1e-5) * g[...] + b[...]
    h_out[...] = hn.astype(jnp.bfloat16)
    z = (jnp.dot(hn, w1h[...], preferred_element_type=jnp.float32)
         + jnp.dot(st[...], w1f[...], preferred_element_type=jnp.float32)
         + b1[...])
    z = _gelu(z)
    logits = jnp.dot(z, w2[...], preferred_element_type=jnp.float32) + b2[...]
    cols = lax.broadcasted_iota(jnp.int32, logits.shape, 1)
    big = jnp.int32(2**30)
    m1 = jnp.max(logits, axis=-1, keepdims=True)
    i1 = jnp.min(jnp.where(logits == m1, cols, big), axis=-1, keepdims=True)
    l2 = jnp.where(cols == i1, -jnp.inf, logits)
    m2 = jnp.max(l2, axis=-1, keepdims=True)
    i2 = jnp.min(jnp.where(l2 == m2, cols, big), axis=-1, keepdims=True)
    w1v = 1.0 / (1.0 + jnp.exp(m2 - m1))
    w2v = 1.0 / (1.0 + jnp.exp(m1 - m2))
    idx_out[...] = jnp.concatenate([i1, i2], axis=-1)
    wp_out[...] = jnp.concatenate([w1v, w2v], axis=-1)


def _router_call(hidden, stage, g2d, b2d, rW1h, rW1f, rb1_2d, rW2p, rb2p):
    return pl.pallas_call(
        _router_body,
        grid=(_B // _TMR,),
        in_specs=[
            pl.BlockSpec((_TMR, _D), lambda i: (i, 0)),
            pl.BlockSpec((_TMR, _NC * _FB), lambda i: (i, 0)),
            pl.BlockSpec((1, _D), lambda i: (0, 0)),
            pl.BlockSpec((1, _D), lambda i: (0, 0)),
            pl.BlockSpec((_D, _RH), lambda i: (0, 0)),
            pl.BlockSpec((_NC * _FB, _RH), lambda i: (0, 0)),
            pl.BlockSpec((1, _RH), lambda i: (0, 0)),
            pl.BlockSpec((_RH, _EPAD), lambda i: (0, 0)),
            pl.BlockSpec((1, _EPAD), lambda i: (0, 0)),
        ],
        out_specs=[
            pl.BlockSpec((_TMR, _D), lambda i: (i, 0)),
            pl.BlockSpec((_TMR, _K), lambda i: (i, 0)),
            pl.BlockSpec((_TMR, _K), lambda i: (i, 0)),
        ],
        out_shape=[
            jax.ShapeDtypeStruct((_B, _D), jnp.bfloat16),
            jax.ShapeDtypeStruct((_B, _K), jnp.int32),
            jax.ShapeDtypeStruct((_B, _K), jnp.float32),
        ],
    )(hidden, stage, g2d, b2d, rW1h, rW1f, rb1_2d, rW2p, rb2p)


def _dispatch_meta(idx):
    idx_flat = idx.reshape(-1).astype(jnp.int32)
    onehot = (idx_flat[:, None] == jnp.arange(_E, dtype=jnp.int32)[None, :])
    csum = jnp.cumsum(onehot.astype(jnp.int32), axis=0)
    counts = csum[-1]
    rank = jnp.take_along_axis(csum, idx_flat[:, None], axis=1)[:, 0] - 1
    offsets = jnp.concatenate(
        [jnp.zeros((1,), jnp.int32), jnp.cumsum(counts)]).astype(jnp.int32)
    pos = offsets[idx_flat] + rank
    perm = jnp.zeros((_KP,), jnp.int32).at[pos].set(
        jnp.arange(_KP, dtype=jnp.int32))
    tok_sorted = perm // _K
    e_sorted = idx_flat[perm]

    starts, ends = offsets[:-1], offsets[1:]
    first_tile = starts // _TM
    last_tile = jnp.where(counts > 0, (ends - 1) // _TM, first_tile)
    ntiles = jnp.where(counts > 0, last_tile - first_tile + 1, 0)
    wu_end = jnp.cumsum(ntiles)
    wu_start = wu_end - ntiles
    total = wu_end[-1]
    j = jnp.arange(_NW, dtype=jnp.int32)
    e_j = jnp.sum((j[:, None] >= wu_end[None, :]).astype(jnp.int32), axis=1)
    valid = j < total
    e_jc = jnp.minimum(e_j, _E - 1)
    tile_j = first_tile[e_jc] + (j - wu_start[e_jc])
    last_e = jnp.max(jnp.where(ntiles > 0, jnp.arange(_E), -1)).astype(jnp.int32)
    wu_tile = jnp.where(valid, tile_j, _NT - 1).astype(jnp.int32)
    wu_expert = jnp.where(valid, e_jc, last_e).astype(jnp.int32)
    wu_lo = jnp.where(valid, starts[wu_expert], 0).astype(jnp.int32)
    wu_hi = jnp.where(valid, ends[wu_expert], 0).astype(jnp.int32)
    return pos, perm, tok_sorted, e_sorted, wu_tile, wu_expert, wu_lo, wu_hi


def _sc_gather(h, stage, tok_sorted):
    mesh = plsc.VectorSubcoreMesh(core_axis_name="c", subcore_axis_name="s")
    rows_w = _KP // _NWORK
    wh = _D // 2
    ws = _NC * _FB // 2

    @functools.partial(
        pl.kernel, mesh=mesh,
        out_type=[jax.ShapeDtypeStruct((_KP, wh), jnp.int32),
                  jax.ShapeDtypeStruct((_KP, ws), jnp.int32)],
        scratch_types=[pltpu.VMEM((rows_w,), jnp.int32),
                       pltpu.VMEM((32,), jnp.int32),
                       pltpu.VMEM((32, wh), jnp.int32),
                       pltpu.VMEM((rows_w, ws), jnp.int32),
                       pltpu.SemaphoreType.DMA],
    )
    def g1(h_hbm, st_hbm, tok_hbm, xh_out, xs_out,
           idx_v, idx32_v, rows_v, srows_v, sem):
        wid = lax.axis_index("s") * 2 + lax.axis_index("c")
        base = wid * rows_w
        pltpu.sync_copy(tok_hbm.at[pl.ds(base, rows_w)], idx_v)
        pltpu.async_copy(st_hbm.at[idx_v], srows_v, sem).wait()
        pltpu.sync_copy(srows_v, xs_out.at[pl.ds(base, rows_w)])

        def hsub(c, carry):
            b = base + c * 32
            pltpu.sync_copy(tok_hbm.at[pl.ds(b, 32)], idx32_v)
            pltpu.async_copy(h_hbm.at[idx32_v], rows_v, sem).wait()
            pltpu.sync_copy(rows_v, xh_out.at[pl.ds(b, 32)])
            return carry
        lax.fori_loop(0, rows_w // 32, hsub, 0)

    return g1(h, stage, tok_sorted)


def _mlp_body(tr, er, lr, hr, xh, xf, ws, w1h, w1f, b1, w2, b2, w3, b3, out):
    j = pl.program_id(0)
    t = tr[j]
    lo = lr[j]
    hi = hr[j]
    rows = t * _TM + lax.broadcasted_iota(jnp.int32, (_TM, 1), 0)
    vmask = (rows >= lo) & (rows < hi)
    z1 = (jnp.dot(xh[...], w1h[0], preferred_element_type=jnp.float32)
          + jnp.dot(xf[...], w1f[0], preferred_element_type=jnp.float32)
          + b1[0])
    h1 = _gelu(z1).astype(jnp.bfloat16)
    h2 = _gelu(jnp.dot(h1, w2[0], preferred_element_type=jnp.float32)
               + b2[0]).astype(jnp.bfloat16)
    y = jnp.dot(h2, w3[0], preferred_element_type=jnp.float32) + b3[0]
    y = y * ws[0, 0, :][:, None]
    contrib = jnp.where(vmask, y, 0.0)
    prev_t = tr[jnp.maximum(j - 1, 0)]
    first = (j == 0) | (t != prev_t)

    @pl.when(first)
    def _():
        out[...] = contrib

    @pl.when(jnp.logical_not(first))
    def _():
        out[...] += contrib


def _mlp_call(meta, xh, xf, ws3, We1h, We1f, be1, We2, be2, We3, be3):
    wu_tile, wu_expert, wu_lo, wu_hi = meta
    grid_spec = pltpu.PrefetchScalarGridSpec(
        num_scalar_prefetch=4,
        grid=(_NW,),
        in_specs=[
            pl.BlockSpec((_TM, _D), lambda j, tr, er, lr, hr: (tr[j], 0)),
            pl.BlockSpec((_TM, _NC * _FB), lambda j, tr, er, lr, hr: (tr[j], 0)),
            pl.BlockSpec((1, 1, _TM), lambda j, tr, er, lr, hr: (tr[j], 0, 0)),
            pl.BlockSpec((1, _D, _H), lambda j, tr, er, lr, hr: (er[j], 0, 0)),
            pl.BlockSpec((1, _NC * _FB, _H),
                         lambda j, tr, er, lr, hr: (er[j], 0, 0)),
            pl.BlockSpec((1, 1, _H), lambda j, tr, er, lr, hr: (er[j], 0, 0)),
            pl.BlockSpec((1, _H, _H), lambda j, tr, er, lr, hr: (er[j], 0, 0)),
            pl.BlockSpec((1, 1, _H), lambda j, tr, er, lr, hr: (er[j], 0, 0)),
            pl.BlockSpec((1, _H, _D), lambda j, tr, er, lr, hr: (er[j], 0, 0)),
            pl.BlockSpec((1, 1, _D), lambda j, tr, er, lr, hr: (er[j], 0, 0)),
        ],
        out_specs=pl.BlockSpec((_TM, _D), lambda j, tr, er, lr, hr: (tr[j], 0)),
    )
    return pl.pallas_call(
        _mlp_body,
        grid_spec=grid_spec,
        out_shape=jax.ShapeDtypeStruct((_KP, _D), jnp.float32),
        compiler_params=pltpu.CompilerParams(
            dimension_semantics=("arbitrary",),
            vmem_limit_bytes=100 * 1024 * 1024,
        ),
    )(wu_tile, wu_expert, wu_lo, wu_hi,
      xh, xf, ws3, We1h, We1f, be1, We2, be2, We3, be3)


def _sc_combine(hidden, ys, pos):
    mesh = plsc.VectorSubcoreMesh(core_axis_name="c", subcore_axis_name="s")
    tok_w = _B // _NWORK
    st = 16
    lanes = _D // 16

    @functools.partial(
        pl.kernel, mesh=mesh,
        out_type=jax.ShapeDtypeStruct((_B, _D), jnp.float32),
        scratch_types=[pltpu.VMEM((2 * st,), jnp.int32),
                       pltpu.VMEM((2 * st, _D), jnp.float32),
                       pltpu.VMEM((st, _D), jnp.float32),
                       pltpu.SemaphoreType.DMA],
    )
    def g2(hid_hbm, ys_hbm, pos_hbm, out_hbm, idx_v, ys_v, hid_v, sem):
        wid = lax.axis_index("s") * 2 + lax.axis_index("c")
        tbase = wid * tok_w

        def sub(c, carry):
            t0 = tbase + c * st
            pltpu.sync_copy(pos_hbm.at[pl.ds(2 * t0, 2 * st)], idx_v)
            cp = pltpu.async_copy(ys_hbm.at[idx_v], ys_v, sem)
            pltpu.sync_copy(hid_hbm.at[pl.ds(t0, st)], hid_v)
            cp.wait()

            def row(r, c2):
                def lane(l, c3):
                    o = l * 16
                    acc = (hid_v[r, pl.ds(o, 16)]
                           + ys_v[2 * r, pl.ds(o, 16)]
                           + ys_v[2 * r + 1, pl.ds(o, 16)])
                    hid_v[r, pl.ds(o, 16)] = acc
                    return c3
                return lax.fori_loop(0, lanes, lane, c2)
            lax.fori_loop(0, st, row, 0)
            pltpu.sync_copy(hid_v, out_hbm.at[pl.ds(t0, st)])
            return carry
        lax.fori_loop(0, tok_w // st, sub, 0)

    return g2(hidden, ys, pos)


def kernel(hidden, feature_bank, expert_bank_idx, ln_gamma, ln_beta,
           rW1, rb1, rW2, rb2, We1, be1, We2, be2, We3, be3, alpha):
    f32 = jnp.float32
    stage = feature_bank.reshape(_B, _NC * _FB)
    g2d = ln_gamma.reshape(1, _D)
    b2d = ln_beta.reshape(1, _D)
    rW1h = rW1[:_D]
    rW1f = rW1[_D:]
    rb1_2d = rb1.reshape(1, _RH)
    rW2p = jnp.zeros((_RH, _EPAD), f32).at[:, :_E].set(rW2)
    rb2p = jnp.full((_EPAD,), -1e30, f32).at[:_E].set(rb2).reshape(1, _EPAD)

    h_n, idx, wp = _router_call(hidden, stage, g2d, b2d, rW1h, rW1f,
                                rb1_2d, rW2p, rb2p)

    (pos, perm, tok_sorted, e_sorted,
     wu_tile, wu_expert, wu_lo, wu_hi) = _dispatch_meta(idx)

    ws3 = (alpha * wp.reshape(-1)[perm]).reshape(_NT, 1, _TM)

    h_i32 = lax.bitcast_convert_type(
        h_n.reshape(_B, _D // 2, 2), jnp.int32)
    st_i32 = lax.bitcast_convert_type(
        stage.astype(jnp.bfloat16).reshape(_B, _NC * _FB // 2, 2), jnp.int32)
    xh_i, xs_i = _sc_gather(h_i32, st_i32, tok_sorted)
    xh_sorted = lax.bitcast_convert_type(xh_i, jnp.bfloat16).reshape(_KP, _D)
    xs_sorted = lax.bitcast_convert_type(
        xs_i, jnp.bfloat16).reshape(_KP, _NC * _FB)

    We1h = We1[:, :_D, :].astype(jnp.bfloat16)
    Wf = We1[:, _D:, :].reshape(_E, _FPE, _FB, _H)
    We1fx = (jnp.zeros((_E, _NC, _FB, _H), f32)
             .at[jnp.arange(_E, dtype=jnp.int32)[:, None], expert_bank_idx]
             .set(Wf).reshape(_E, _NC * _FB, _H).astype(jnp.bfloat16))
    ys = _mlp_call((wu_tile, wu_expert, wu_lo, wu_hi),
                   xh_sorted, xs_sorted, ws3,
                   We1h, We1fx, be1.reshape(_E, 1, _H),
                   We2.astype(jnp.bfloat16), be2.reshape(_E, 1, _H),
                   We3.astype(jnp.bfloat16), be3.reshape(_E, 1, _D))

    return _sc_combine(hidden, ys, pos)


# speedup vs baseline: 1.5199x; 1.5199x over previous
import functools

import jax
import jax.numpy as jnp
from jax import lax
from jax.experimental import pallas as pl
from jax.experimental.pallas import tpu as pltpu
from jax.experimental.pallas import tpu_sc as plsc

_B = 2048
_D = 2048
_E = 8
_NC = 16
_FB = 16
_FPE = 2
_H = 1024
_RH = 1024
_K = 2
_EPAD = 128

_TMR = 256
_TM = 128
_KP = _B * _K
_NT = _KP // _TM
_NW = _NT + _E - 1
_NWORK = 32


def _gelu(x):
    return 0.5 * x * (1.0 + lax.erf(x * 0.7071067811865476))


def _router_body(hid, st, g, b, w1h, w1f, b1, w2, b2, h_out, idx_out, wp_out):
    x = hid[...]
    mu = jnp.mean(x, axis=-1, keepdims=True)
    xc = x - mu
    var = jnp.mean(xc * xc, axis=-1, keepdims=True)
    hn = xc * lax.rsqrt(var + 1e-5) * g[...] + b[...]
    h_out[...] = hn
    z = (jnp.dot(hn, w1h[...], preferred_element_type=jnp.float32)
         + jnp.dot(st[...], w1f[...], preferred_element_type=jnp.float32)
         + b1[...])
    z = _gelu(z)
    logits = jnp.dot(z, w2[...], preferred_element_type=jnp.float32) + b2[...]
    cols = lax.broadcasted_iota(jnp.int32, logits.shape, 1)
    big = jnp.int32(2**30)
    m1 = jnp.max(logits, axis=-1, keepdims=True)
    i1 = jnp.min(jnp.where(logits == m1, cols, big), axis=-1, keepdims=True)
    l2 = jnp.where(cols == i1, -jnp.inf, logits)
    m2 = jnp.max(l2, axis=-1, keepdims=True)
    i2 = jnp.min(jnp.where(l2 == m2, cols, big), axis=-1, keepdims=True)
    w1v = 1.0 / (1.0 + jnp.exp(m2 - m1))
    w2v = 1.0 / (1.0 + jnp.exp(m1 - m2))
    idx_out[...] = jnp.concatenate([i1, i2], axis=-1)
    wp_out[...] = jnp.concatenate([w1v, w2v], axis=-1)


def _router_call(hidden, stage, g2d, b2d, rW1h, rW1f, rb1_2d, rW2p, rb2p):
    return pl.pallas_call(
        _router_body,
        grid=(_B // _TMR,),
        in_specs=[
            pl.BlockSpec((_TMR, _D), lambda i: (i, 0)),
            pl.BlockSpec((_TMR, _NC * _FB), lambda i: (i, 0)),
            pl.BlockSpec((1, _D), lambda i: (0, 0)),
            pl.BlockSpec((1, _D), lambda i: (0, 0)),
            pl.BlockSpec((_D, _RH), lambda i: (0, 0)),
            pl.BlockSpec((_NC * _FB, _RH), lambda i: (0, 0)),
            pl.BlockSpec((1, _RH), lambda i: (0, 0)),
            pl.BlockSpec((_RH, _EPAD), lambda i: (0, 0)),
            pl.BlockSpec((1, _EPAD), lambda i: (0, 0)),
        ],
        out_specs=[
            pl.BlockSpec((_TMR, _D), lambda i: (i, 0)),
            pl.BlockSpec((_TMR, _K), lambda i: (i, 0)),
            pl.BlockSpec((_TMR, _K), lambda i: (i, 0)),
        ],
        out_shape=[
            jax.ShapeDtypeStruct((_B, _D), jnp.float32),
            jax.ShapeDtypeStruct((_B, _K), jnp.int32),
            jax.ShapeDtypeStruct((_B, _K), jnp.float32),
        ],
    )(hidden, stage, g2d, b2d, rW1h, rW1f, rb1_2d, rW2p, rb2p)


def _dispatch_meta(idx):
    idx_flat = idx.reshape(-1).astype(jnp.int32)
    onehot = (idx_flat[:, None] == jnp.arange(_E, dtype=jnp.int32)[None, :])
    csum = jnp.cumsum(onehot.astype(jnp.int32), axis=0)
    counts = csum[-1]
    rank = jnp.take_along_axis(csum, idx_flat[:, None], axis=1)[:, 0] - 1
    offsets = jnp.concatenate(
        [jnp.zeros((1,), jnp.int32), jnp.cumsum(counts)]).astype(jnp.int32)
    pos = offsets[idx_flat] + rank
    perm = jnp.zeros((_KP,), jnp.int32).at[pos].set(
        jnp.arange(_KP, dtype=jnp.int32))
    tok_sorted = perm // _K
    e_sorted = idx_flat[perm]

    starts, ends = offsets[:-1], offsets[1:]
    first_tile = starts // _TM
    last_tile = jnp.where(counts > 0, (ends - 1) // _TM, first_tile)
    ntiles = jnp.where(counts > 0, last_tile - first_tile + 1, 0)
    wu_end = jnp.cumsum(ntiles)
    wu_start = wu_end - ntiles
    total = wu_end[-1]
    j = jnp.arange(_NW, dtype=jnp.int32)
    e_j = jnp.sum((j[:, None] >= wu_end[None, :]).astype(jnp.int32), axis=1)
    valid = j < total
    e_jc = jnp.minimum(e_j, _E - 1)
    tile_j = first_tile[e_jc] + (j - wu_start[e_jc])
    last_e = jnp.max(jnp.where(ntiles > 0, jnp.arange(_E), -1)).astype(jnp.int32)
    wu_tile = jnp.where(valid, tile_j, _NT - 1).astype(jnp.int32)
    wu_expert = jnp.where(valid, e_jc, last_e).astype(jnp.int32)
    wu_lo = jnp.where(valid, starts[wu_expert], 0).astype(jnp.int32)
    wu_hi = jnp.where(valid, ends[wu_expert], 0).astype(jnp.int32)
    return pos, perm, tok_sorted, e_sorted, wu_tile, wu_expert, wu_lo, wu_hi


def _sc_gather(h, stage, tok_sorted):
    mesh = plsc.VectorSubcoreMesh(core_axis_name="c", subcore_axis_name="s")
    rows_w = _KP // _NWORK
    sf = _NC * _FB

    @functools.partial(
        pl.kernel, mesh=mesh,
        out_type=[jax.ShapeDtypeStruct((_KP, _D), jnp.float32),
                  jax.ShapeDtypeStruct((_KP, sf), jnp.float32)],
        scratch_types=[pltpu.VMEM((rows_w,), jnp.int32),
                       pltpu.VMEM((32,), jnp.int32),
                       pltpu.VMEM((32, _D), jnp.float32),
                       pltpu.VMEM((rows_w, sf), jnp.float32),
                       pltpu.SemaphoreType.DMA],
    )
    def g1(h_hbm, st_hbm, tok_hbm, xh_out, xs_out,
           idx_v, idx32_v, rows_v, srows_v, sem):
        wid = lax.axis_index("s") * 2 + lax.axis_index("c")
        base = wid * rows_w
        pltpu.sync_copy(tok_hbm.at[pl.ds(base, rows_w)], idx_v)
        pltpu.async_copy(st_hbm.at[idx_v], srows_v, sem).wait()
        pltpu.sync_copy(srows_v, xs_out.at[pl.ds(base, rows_w)])

        def hsub(c, carry):
            b = base + c * 32
            pltpu.sync_copy(tok_hbm.at[pl.ds(b, 32)], idx32_v)
            pltpu.async_copy(h_hbm.at[idx32_v], rows_v, sem).wait()
            pltpu.sync_copy(rows_v, xh_out.at[pl.ds(b, 32)])
            return carry
        lax.fori_loop(0, rows_w // 32, hsub, 0)

    return g1(h, stage, tok_sorted)


def _mlp_body(tr, er, lr, hr, xh, xf, ws, w1h, w1f, b1, w2, b2, w3, b3, out):
    j = pl.program_id(0)
    t = tr[j]
    lo = lr[j]
    hi = hr[j]
    rows = t * _TM + lax.broadcasted_iota(jnp.int32, (_TM, 1), 0)
    vmask = (rows >= lo) & (rows < hi)
    z1 = (jnp.dot(xh[...].astype(jnp.bfloat16), w1h[0],
                  preferred_element_type=jnp.float32)
          + jnp.dot(xf[...].astype(jnp.bfloat16), w1f[0],
                    preferred_element_type=jnp.float32)
          + b1[0])
    h1 = _gelu(z1).astype(jnp.bfloat16)
    h2 = _gelu(jnp.dot(h1, w2[0], preferred_element_type=jnp.float32)
               + b2[0]).astype(jnp.bfloat16)
    y = jnp.dot(h2, w3[0], preferred_element_type=jnp.float32) + b3[0]
    y = y * ws[0, 0, :][:, None]
    contrib = jnp.where(vmask, y, 0.0)
    prev_t = tr[jnp.maximum(j - 1, 0)]
    first = (j == 0) | (t != prev_t)

    @pl.when(first)
    def _():
        out[...] = contrib

    @pl.when(jnp.logical_not(first))
    def _():
        out[...] += contrib


def _mlp_call(meta, xh, xf, ws3, We1h, We1f, be1, We2, be2, We3, be3):
    wu_tile, wu_expert, wu_lo, wu_hi = meta
    grid_spec = pltpu.PrefetchScalarGridSpec(
        num_scalar_prefetch=4,
        grid=(_NW,),
        in_specs=[
            pl.BlockSpec((_TM, _D), lambda j, tr, er, lr, hr: (tr[j], 0)),
            pl.BlockSpec((_TM, _NC * _FB), lambda j, tr, er, lr, hr: (tr[j], 0)),
            pl.BlockSpec((1, 1, _TM), lambda j, tr, er, lr, hr: (tr[j], 0, 0)),
            pl.BlockSpec((1, _D, _H), lambda j, tr, er, lr, hr: (er[j], 0, 0)),
            pl.BlockSpec((1, _NC * _FB, _H),
                         lambda j, tr, er, lr, hr: (er[j], 0, 0)),
            pl.BlockSpec((1, 1, _H), lambda j, tr, er, lr, hr: (er[j], 0, 0)),
            pl.BlockSpec((1, _H, _H), lambda j, tr, er, lr, hr: (er[j], 0, 0)),
            pl.BlockSpec((1, 1, _H), lambda j, tr, er, lr, hr: (er[j], 0, 0)),
            pl.BlockSpec((1, _H, _D), lambda j, tr, er, lr, hr: (er[j], 0, 0)),
            pl.BlockSpec((1, 1, _D), lambda j, tr, er, lr, hr: (er[j], 0, 0)),
        ],
        out_specs=pl.BlockSpec((_TM, _D), lambda j, tr, er, lr, hr: (tr[j], 0)),
    )
    return pl.pallas_call(
        _mlp_body,
        grid_spec=grid_spec,
        out_shape=jax.ShapeDtypeStruct((_KP, _D), jnp.float32),
        compiler_params=pltpu.CompilerParams(
            dimension_semantics=("arbitrary",),
            vmem_limit_bytes=100 * 1024 * 1024,
        ),
    )(wu_tile, wu_expert, wu_lo, wu_hi,
      xh, xf, ws3, We1h, We1f, be1, We2, be2, We3, be3)


def _sc_combine(hidden, ys, pos):
    mesh = plsc.VectorSubcoreMesh(core_axis_name="c", subcore_axis_name="s")
    tok_w = _B // _NWORK
    st = 16
    lanes = _D // 16

    @functools.partial(
        pl.kernel, mesh=mesh,
        out_type=jax.ShapeDtypeStruct((_B, _D), jnp.float32),
        scratch_types=[pltpu.VMEM((2 * st,), jnp.int32),
                       pltpu.VMEM((2 * st, _D), jnp.float32),
                       pltpu.VMEM((st, _D), jnp.float32),
                       pltpu.SemaphoreType.DMA],
    )
    def g2(hid_hbm, ys_hbm, pos_hbm, out_hbm, idx_v, ys_v, hid_v, sem):
        wid = lax.axis_index("s") * 2 + lax.axis_index("c")
        tbase = wid * tok_w

        def sub(c, carry):
            t0 = tbase + c * st
            pltpu.sync_copy(pos_hbm.at[pl.ds(2 * t0, 2 * st)], idx_v)
            cp = pltpu.async_copy(ys_hbm.at[idx_v], ys_v, sem)
            pltpu.sync_copy(hid_hbm.at[pl.ds(t0, st)], hid_v)
            cp.wait()

            def row(r, c2):
                def lane(l, c3):
                    o = l * 16
                    acc = (hid_v[r, pl.ds(o, 16)]
                           + ys_v[2 * r, pl.ds(o, 16)]
                           + ys_v[2 * r + 1, pl.ds(o, 16)])
                    hid_v[r, pl.ds(o, 16)] = acc
                    return c3
                return lax.fori_loop(0, lanes, lane, c2)
            lax.fori_loop(0, st, row, 0)
            pltpu.sync_copy(hid_v, out_hbm.at[pl.ds(t0, st)])
            return carry
        lax.fori_loop(0, tok_w // st, sub, 0)

    return g2(hidden, ys, pos)


def kernel(hidden, feature_bank, expert_bank_idx, ln_gamma, ln_beta,
           rW1, rb1, rW2, rb2, We1, be1, We2, be2, We3, be3, alpha):
    f32 = jnp.float32
    stage = feature_bank.reshape(_B, _NC * _FB)
    g2d = ln_gamma.reshape(1, _D)
    b2d = ln_beta.reshape(1, _D)
    rW1h = rW1[:_D]
    rW1f = rW1[_D:]
    rb1_2d = rb1.reshape(1, _RH)
    rW2p = jnp.zeros((_RH, _EPAD), f32).at[:, :_E].set(rW2)
    rb2p = jnp.full((_EPAD,), -1e30, f32).at[:_E].set(rb2).reshape(1, _EPAD)

    h_n, idx, wp = _router_call(hidden, stage, g2d, b2d, rW1h, rW1f,
                                rb1_2d, rW2p, rb2p)

    (pos, perm, tok_sorted, e_sorted,
     wu_tile, wu_expert, wu_lo, wu_hi) = _dispatch_meta(idx)

    ws3 = (alpha * wp.reshape(-1)[perm]).reshape(_NT, 1, _TM)

    xh_sorted, xs_sorted = _sc_gather(h_n, stage, tok_sorted)

    We1h = We1[:, :_D, :].astype(jnp.bfloat16)
    Wf = We1[:, _D:, :].reshape(_E, _FPE, _FB, _H)
    We1fx = (jnp.zeros((_E, _NC, _FB, _H), f32)
             .at[jnp.arange(_E, dtype=jnp.int32)[:, None], expert_bank_idx]
             .set(Wf).reshape(_E, _NC * _FB, _H).astype(jnp.bfloat16))
    ys = _mlp_call((wu_tile, wu_expert, wu_lo, wu_hi),
                   xh_sorted, xs_sorted, ws3,
                   We1h, We1fx, be1.reshape(_E, 1, _H),
                   We2.astype(jnp.bfloat16), be2.reshape(_E, 1, _H),
                   We3.astype(jnp.bfloat16), be3.reshape(_E, 1, _D))

    return _sc_combine(hidden, ys, pos)


# speedup vs baseline: 1.6045x; 1.0557x over previous
import functools

import jax
import jax.numpy as jnp
from jax import lax
from jax.experimental import pallas as pl
from jax.experimental.pallas import tpu as pltpu
from jax.experimental.pallas import tpu_sc as plsc

_B = 2048
_D = 2048
_E = 8
_NC = 16
_FB = 16
_FPE = 2
_H = 1024
_RH = 1024
_K = 2
_EPAD = 128

_TMR = 256
_TM = 128
_KP = _B * _K
_NT = _KP // _TM
_NW = _NT + _E - 1
_NWORK = 32


def _gelu(x):
    return 0.5 * x * (1.0 + lax.erf(x * 0.7071067811865476))


def _router_body(hid, st, g, b, w1h, w1f, b1, w2, b2, h_out, idx_out, wp_out):
    x = hid[...]
    mu = jnp.mean(x, axis=-1, keepdims=True)
    xc = x - mu
    var = jnp.mean(xc * xc, axis=-1, keepdims=True)
    hn = xc * lax.rsqrt(var + 1e-5) * g[...] + b[...]
    h_out[...] = hn
    z = (jnp.dot(hn, w1h[...], preferred_element_type=jnp.float32)
         + jnp.dot(st[...], w1f[...], preferred_element_type=jnp.float32)
         + b1[...])
    z = _gelu(z)
    logits = jnp.dot(z, w2[...], preferred_element_type=jnp.float32) + b2[...]
    cols = lax.broadcasted_iota(jnp.int32, logits.shape, 1)
    big = jnp.int32(2**30)
    m1 = jnp.max(logits, axis=-1, keepdims=True)
    i1 = jnp.min(jnp.where(logits == m1, cols, big), axis=-1, keepdims=True)
    l2 = jnp.where(cols == i1, -jnp.inf, logits)
    m2 = jnp.max(l2, axis=-1, keepdims=True)
    i2 = jnp.min(jnp.where(l2 == m2, cols, big), axis=-1, keepdims=True)
    w1v = 1.0 / (1.0 + jnp.exp(m2 - m1))
    w2v = 1.0 / (1.0 + jnp.exp(m1 - m2))
    idx_out[...] = jnp.concatenate([i1, i2], axis=-1)
    wp_out[...] = jnp.concatenate([w1v, w2v], axis=-1)


def _router_call(hidden, stage, g2d, b2d, rW1h, rW1f, rb1_2d, rW2p, rb2p):
    return pl.pallas_call(
        _router_body,
        grid=(_B // _TMR,),
        in_specs=[
            pl.BlockSpec((_TMR, _D), lambda i: (i, 0)),
            pl.BlockSpec((_TMR, _NC * _FB), lambda i: (i, 0)),
            pl.BlockSpec((1, _D), lambda i: (0, 0)),
            pl.BlockSpec((1, _D), lambda i: (0, 0)),
            pl.BlockSpec((_D, _RH), lambda i: (0, 0)),
            pl.BlockSpec((_NC * _FB, _RH), lambda i: (0, 0)),
            pl.BlockSpec((1, _RH), lambda i: (0, 0)),
            pl.BlockSpec((_RH, _EPAD), lambda i: (0, 0)),
            pl.BlockSpec((1, _EPAD), lambda i: (0, 0)),
        ],
        out_specs=[
            pl.BlockSpec((_TMR, _D), lambda i: (i, 0)),
            pl.BlockSpec((_TMR, _K), lambda i: (i, 0)),
            pl.BlockSpec((_TMR, _K), lambda i: (i, 0)),
        ],
        out_shape=[
            jax.ShapeDtypeStruct((_B, _D), jnp.float32),
            jax.ShapeDtypeStruct((_B, _K), jnp.int32),
            jax.ShapeDtypeStruct((_B, _K), jnp.float32),
        ],
    )(hidden, stage, g2d, b2d, rW1h, rW1f, rb1_2d, rW2p, rb2p)


def _dispatch_meta(idx):
    idx_flat = idx.reshape(-1).astype(jnp.int32)
    onehot = (idx_flat[:, None] == jnp.arange(_E, dtype=jnp.int32)[None, :])
    csum = jnp.cumsum(onehot.astype(jnp.int32), axis=0)
    counts = csum[-1]
    rank = jnp.take_along_axis(csum, idx_flat[:, None], axis=1)[:, 0] - 1
    offsets = jnp.concatenate(
        [jnp.zeros((1,), jnp.int32), jnp.cumsum(counts)]).astype(jnp.int32)
    pos = offsets[idx_flat] + rank
    perm = jnp.zeros((_KP,), jnp.int32).at[pos].set(
        jnp.arange(_KP, dtype=jnp.int32))
    tok_sorted = perm // _K
    e_sorted = idx_flat[perm]

    starts, ends = offsets[:-1], offsets[1:]
    first_tile = starts // _TM
    last_tile = jnp.where(counts > 0, (ends - 1) // _TM, first_tile)
    ntiles = jnp.where(counts > 0, last_tile - first_tile + 1, 0)
    wu_end = jnp.cumsum(ntiles)
    wu_start = wu_end - ntiles
    total = wu_end[-1]
    j = jnp.arange(_NW, dtype=jnp.int32)
    e_j = jnp.sum((j[:, None] >= wu_end[None, :]).astype(jnp.int32), axis=1)
    valid = j < total
    e_jc = jnp.minimum(e_j, _E - 1)
    tile_j = first_tile[e_jc] + (j - wu_start[e_jc])
    last_e = jnp.max(jnp.where(ntiles > 0, jnp.arange(_E), -1)).astype(jnp.int32)
    wu_tile = jnp.where(valid, tile_j, _NT - 1).astype(jnp.int32)
    wu_expert = jnp.where(valid, e_jc, last_e).astype(jnp.int32)
    wu_lo = jnp.where(valid, starts[wu_expert], 0).astype(jnp.int32)
    wu_hi = jnp.where(valid, ends[wu_expert], 0).astype(jnp.int32)
    return pos, perm, tok_sorted, e_sorted, wu_tile, wu_expert, wu_lo, wu_hi


def _sc_gather(h, stage, tok_sorted):
    mesh = plsc.VectorSubcoreMesh(core_axis_name="c", subcore_axis_name="s")
    rows_w = _KP // _NWORK
    sf = _NC * _FB

    @functools.partial(
        pl.kernel, mesh=mesh,
        out_type=[jax.ShapeDtypeStruct((_KP, _D), jnp.float32),
                  jax.ShapeDtypeStruct((_KP, sf), jnp.float32)],
        scratch_types=[pltpu.VMEM((rows_w,), jnp.int32),
                       pltpu.VMEM((32,), jnp.int32),
                       pltpu.VMEM((32, _D), jnp.float32),
                       pltpu.VMEM((rows_w, sf), jnp.float32),
                       pltpu.SemaphoreType.DMA],
    )
    def g1(h_hbm, st_hbm, tok_hbm, xh_out, xs_out,
           idx_v, idx32_v, rows_v, srows_v, sem):
        wid = lax.axis_index("s") * 2 + lax.axis_index("c")
        base = wid * rows_w
        pltpu.sync_copy(tok_hbm.at[pl.ds(base, rows_w)], idx_v)
        pltpu.async_copy(st_hbm.at[idx_v], srows_v, sem).wait()
        pltpu.sync_copy(srows_v, xs_out.at[pl.ds(base, rows_w)])

        def hsub(c, carry):
            b = base + c * 32
            pltpu.sync_copy(tok_hbm.at[pl.ds(b, 32)], idx32_v)
            pltpu.async_copy(h_hbm.at[idx32_v], rows_v, sem).wait()
            pltpu.sync_copy(rows_v, xh_out.at[pl.ds(b, 32)])
            return carry
        lax.fori_loop(0, rows_w // 32, hsub, 0)

    return g1(h, stage, tok_sorted)


def _mlp_body(tr, er, lr, hr, xh, xf, ws, w1h, w1f, b1, w2, b2, w3, b3, out):
    j = pl.program_id(0)
    t = tr[j]
    lo = lr[j]
    hi = hr[j]
    rows = t * _TM + lax.broadcasted_iota(jnp.int32, (_TM, 1), 0)
    vmask = (rows >= lo) & (rows < hi)
    z1 = (jnp.dot(xh[...], w1h[0], preferred_element_type=jnp.float32)
          + jnp.dot(xf[...], w1f[0], preferred_element_type=jnp.float32)
          + b1[0])
    h1 = _gelu(z1)
    h2 = _gelu(jnp.dot(h1, w2[0], preferred_element_type=jnp.float32) + b2[0])
    y = jnp.dot(h2, w3[0], preferred_element_type=jnp.float32) + b3[0]
    y = y * ws[0, 0, :][:, None]
    contrib = jnp.where(vmask, y, 0.0)
    prev_t = tr[jnp.maximum(j - 1, 0)]
    first = (j == 0) | (t != prev_t)

    @pl.when(first)
    def _():
        out[...] = contrib

    @pl.when(jnp.logical_not(first))
    def _():
        out[...] += contrib


def _mlp_call(meta, xh, xf, ws3, We1h, We1f, be1, We2, be2, We3, be3):
    wu_tile, wu_expert, wu_lo, wu_hi = meta
    grid_spec = pltpu.PrefetchScalarGridSpec(
        num_scalar_prefetch=4,
        grid=(_NW,),
        in_specs=[
            pl.BlockSpec((_TM, _D), lambda j, tr, er, lr, hr: (tr[j], 0)),
            pl.BlockSpec((_TM, _NC * _FB), lambda j, tr, er, lr, hr: (tr[j], 0)),
            pl.BlockSpec((1, 1, _TM), lambda j, tr, er, lr, hr: (tr[j], 0, 0)),
            pl.BlockSpec((1, _D, _H), lambda j, tr, er, lr, hr: (er[j], 0, 0)),
            pl.BlockSpec((1, _NC * _FB, _H),
                         lambda j, tr, er, lr, hr: (er[j], 0, 0)),
            pl.BlockSpec((1, 1, _H), lambda j, tr, er, lr, hr: (er[j], 0, 0)),
            pl.BlockSpec((1, _H, _H), lambda j, tr, er, lr, hr: (er[j], 0, 0)),
            pl.BlockSpec((1, 1, _H), lambda j, tr, er, lr, hr: (er[j], 0, 0)),
            pl.BlockSpec((1, _H, _D), lambda j, tr, er, lr, hr: (er[j], 0, 0)),
            pl.BlockSpec((1, 1, _D), lambda j, tr, er, lr, hr: (er[j], 0, 0)),
        ],
        out_specs=pl.BlockSpec((_TM, _D), lambda j, tr, er, lr, hr: (tr[j], 0)),
    )
    return pl.pallas_call(
        _mlp_body,
        grid_spec=grid_spec,
        out_shape=jax.ShapeDtypeStruct((_KP, _D), jnp.float32),
        compiler_params=pltpu.CompilerParams(
            dimension_semantics=("arbitrary",),
            vmem_limit_bytes=100 * 1024 * 1024,
        ),
    )(wu_tile, wu_expert, wu_lo, wu_hi,
      xh, xf, ws3, We1h, We1f, be1, We2, be2, We3, be3)


def _sc_combine(hidden, ys, pos):
    mesh = plsc.VectorSubcoreMesh(core_axis_name="c", subcore_axis_name="s")
    tok_w = _B // _NWORK
    st = 16
    lanes = _D // 16

    @functools.partial(
        pl.kernel, mesh=mesh,
        out_type=jax.ShapeDtypeStruct((_B, _D), jnp.float32),
        scratch_types=[pltpu.VMEM((2 * st,), jnp.int32),
                       pltpu.VMEM((2 * st, _D), jnp.float32),
                       pltpu.VMEM((st, _D), jnp.float32),
                       pltpu.SemaphoreType.DMA],
    )
    def g2(hid_hbm, ys_hbm, pos_hbm, out_hbm, idx_v, ys_v, hid_v, sem):
        wid = lax.axis_index("s") * 2 + lax.axis_index("c")
        tbase = wid * tok_w

        def sub(c, carry):
            t0 = tbase + c * st
            pltpu.sync_copy(pos_hbm.at[pl.ds(2 * t0, 2 * st)], idx_v)
            cp = pltpu.async_copy(ys_hbm.at[idx_v], ys_v, sem)
            pltpu.sync_copy(hid_hbm.at[pl.ds(t0, st)], hid_v)
            cp.wait()

            def row(r, c2):
                def lane(l, c3):
                    o = l * 16
                    acc = (hid_v[r, pl.ds(o, 16)]
                           + ys_v[2 * r, pl.ds(o, 16)]
                           + ys_v[2 * r + 1, pl.ds(o, 16)])
                    hid_v[r, pl.ds(o, 16)] = acc
                    return c3
                return lax.fori_loop(0, lanes, lane, c2)
            lax.fori_loop(0, st, row, 0)
            pltpu.sync_copy(hid_v, out_hbm.at[pl.ds(t0, st)])
            return carry
        lax.fori_loop(0, tok_w // st, sub, 0)

    return g2(hidden, ys, pos)


def kernel(hidden, feature_bank, expert_bank_idx, ln_gamma, ln_beta,
           rW1, rb1, rW2, rb2, We1, be1, We2, be2, We3, be3, alpha):
    f32 = jnp.float32
    stage = feature_bank.reshape(_B, _NC * _FB)
    g2d = ln_gamma.reshape(1, _D)
    b2d = ln_beta.reshape(1, _D)
    rW1h = rW1[:_D]
    rW1f = rW1[_D:]
    rb1_2d = rb1.reshape(1, _RH)
    rW2p = jnp.zeros((_RH, _EPAD), f32).at[:, :_E].set(rW2)
    rb2p = jnp.full((_EPAD,), -1e30, f32).at[:_E].set(rb2).reshape(1, _EPAD)

    h_n, idx, wp = _router_call(hidden, stage, g2d, b2d, rW1h, rW1f,
                                rb1_2d, rW2p, rb2p)

    (pos, perm, tok_sorted, e_sorted,
     wu_tile, wu_expert, wu_lo, wu_hi) = _dispatch_meta(idx)

    ws3 = (alpha * wp.reshape(-1)[perm]).reshape(_NT, 1, _TM)

    xh_sorted, xs_sorted = _sc_gather(h_n, stage, tok_sorted)

    We1h = We1[:, :_D, :]
    Wf = We1[:, _D:, :].reshape(_E, _FPE, _FB, _H)
    We1fx = (jnp.zeros((_E, _NC, _FB, _H), f32)
             .at[jnp.arange(_E, dtype=jnp.int32)[:, None], expert_bank_idx]
             .set(Wf).reshape(_E, _NC * _FB, _H))
    ys = _mlp_call((wu_tile, wu_expert, wu_lo, wu_hi),
                   xh_sorted, xs_sorted, ws3,
                   We1h, We1fx, be1.reshape(_E, 1, _H), We2,
                   be2.reshape(_E, 1, _H), We3, be3.reshape(_E, 1, _D))

    return _sc_combine(hidden, ys, pos)


# speedup vs baseline: 1.8744x; 1.1682x over previous
import functools

import jax
import jax.numpy as jnp
from jax import lax
from jax.experimental import pallas as pl
from jax.experimental.pallas import tpu as pltpu
from jax.experimental.pallas import tpu_sc as plsc

_B = 2048
_D = 2048
_E = 8
_NC = 16
_FB = 16
_FPE = 2
_H = 1024
_RH = 1024
_K = 2
_EPAD = 128

_TMR = 256
_TM = 256
_KP = _B * _K
_NT = _KP // _TM
_NW = _NT + _E - 1
_NWORK = 32


def _gelu(x):
    return 0.5 * x * (1.0 + lax.erf(x * 0.7071067811865476))


def _router_body(hid, st, g, b, w1h, w1f, b1, w2, b2, h_out, idx_out, wp_out):
    x = hid[...]
    mu = jnp.mean(x, axis=-1, keepdims=True)
    xc = x - mu
    var = jnp.mean(xc * xc, axis=-1, keepdims=True)
    hn = xc * lax.rsqrt(var + 1e-5) * g[...] + b[...]
    h_out[...] = hn
    z = (jnp.dot(hn, w1h[...], preferred_element_type=jnp.float32)
         + jnp.dot(st[...], w1f[...], preferred_element_type=jnp.float32)
         + b1[...])
    z = _gelu(z)
    logits = jnp.dot(z, w2[...], preferred_element_type=jnp.float32) + b2[...]
    cols = lax.broadcasted_iota(jnp.int32, logits.shape, 1)
    big = jnp.int32(2**30)
    m1 = jnp.max(logits, axis=-1, keepdims=True)
    i1 = jnp.min(jnp.where(logits == m1, cols, big), axis=-1, keepdims=True)
    l2 = jnp.where(cols == i1, -jnp.inf, logits)
    m2 = jnp.max(l2, axis=-1, keepdims=True)
    i2 = jnp.min(jnp.where(l2 == m2, cols, big), axis=-1, keepdims=True)
    w1v = 1.0 / (1.0 + jnp.exp(m2 - m1))
    w2v = 1.0 / (1.0 + jnp.exp(m1 - m2))
    idx_out[...] = jnp.concatenate([i1, i2], axis=-1)
    wp_out[...] = jnp.concatenate([w1v, w2v], axis=-1)


def _router_call(hidden, stage, g2d, b2d, rW1, rb1_2d, rW2p, rb2p):
    return pl.pallas_call(
        _router_body,
        grid=(_B // _TMR,),
        in_specs=[
            pl.BlockSpec((_TMR, _D), lambda i: (i, 0)),
            pl.BlockSpec((_TMR, _NC * _FB), lambda i: (i, 0)),
            pl.BlockSpec((1, _D), lambda i: (0, 0)),
            pl.BlockSpec((1, _D), lambda i: (0, 0)),
            pl.BlockSpec((_D, _RH), lambda i: (0, 0)),
            pl.BlockSpec((_NC * _FB, _RH),
                         lambda i: (_D // (_NC * _FB), 0)),
            pl.BlockSpec((1, _RH), lambda i: (0, 0)),
            pl.BlockSpec((_RH, _EPAD), lambda i: (0, 0)),
            pl.BlockSpec((1, _EPAD), lambda i: (0, 0)),
        ],
        out_specs=[
            pl.BlockSpec((_TMR, _D), lambda i: (i, 0)),
            pl.BlockSpec((_TMR, _K), lambda i: (i, 0)),
            pl.BlockSpec((_TMR, _K), lambda i: (i, 0)),
        ],
        out_shape=[
            jax.ShapeDtypeStruct((_B, _D), jnp.float32),
            jax.ShapeDtypeStruct((_B, _K), jnp.int32),
            jax.ShapeDtypeStruct((_B, _K), jnp.float32),
        ],
    )(hidden, stage, g2d, b2d, rW1, rW1, rb1_2d, rW2p, rb2p)


def _dispatch_meta(idx):
    idx_flat = idx.reshape(-1).astype(jnp.int32)
    onehot = (idx_flat[:, None] == jnp.arange(_E, dtype=jnp.int32)[None, :])
    csum = jnp.cumsum(onehot.astype(jnp.int32), axis=0)
    counts = csum[-1]
    rank = jnp.take_along_axis(csum, idx_flat[:, None], axis=1)[:, 0] - 1
    offsets = jnp.concatenate(
        [jnp.zeros((1,), jnp.int32), jnp.cumsum(counts)]).astype(jnp.int32)
    pos = offsets[idx_flat] + rank
    perm = jnp.zeros((_KP,), jnp.int32).at[pos].set(
        jnp.arange(_KP, dtype=jnp.int32))
    tok_sorted = perm // _K
    e_sorted = idx_flat[perm]

    starts, ends = offsets[:-1], offsets[1:]
    first_tile = starts // _TM
    last_tile = jnp.where(counts > 0, (ends - 1) // _TM, first_tile)
    ntiles = jnp.where(counts > 0, last_tile - first_tile + 1, 0)
    wu_end = jnp.cumsum(ntiles)
    wu_start = wu_end - ntiles
    total = wu_end[-1]
    j = jnp.arange(_NW, dtype=jnp.int32)
    e_j = jnp.sum((j[:, None] >= wu_end[None, :]).astype(jnp.int32), axis=1)
    valid = j < total
    e_jc = jnp.minimum(e_j, _E - 1)
    tile_j = first_tile[e_jc] + (j - wu_start[e_jc])
    last_e = jnp.max(jnp.where(ntiles > 0, jnp.arange(_E), -1)).astype(jnp.int32)
    wu_tile = jnp.where(valid, tile_j, _NT - 1).astype(jnp.int32)
    wu_expert = jnp.where(valid, e_jc, last_e).astype(jnp.int32)
    wu_lo = jnp.where(valid, starts[wu_expert], 0).astype(jnp.int32)
    wu_hi = jnp.where(valid, ends[wu_expert], 0).astype(jnp.int32)
    return pos, perm, tok_sorted, e_sorted, wu_tile, wu_expert, wu_lo, wu_hi


def _sc_gather(h, stage, tok_sorted):
    mesh = plsc.VectorSubcoreMesh(core_axis_name="c", subcore_axis_name="s")
    rows_w = _KP // _NWORK
    sf = _NC * _FB

    @functools.partial(
        pl.kernel, mesh=mesh,
        out_type=[jax.ShapeDtypeStruct((_KP, _D), jnp.float32),
                  jax.ShapeDtypeStruct((_KP, sf), jnp.float32)],
        scratch_types=[pltpu.VMEM((rows_w,), jnp.int32),
                       pltpu.VMEM((32,), jnp.int32),
                       pltpu.VMEM((32, _D), jnp.float32),
                       pltpu.VMEM((rows_w, sf), jnp.float32),
                       pltpu.SemaphoreType.DMA],
    )
    def g1(h_hbm, st_hbm, tok_hbm, xh_out, xs_out,
           idx_v, idx32_v, rows_v, srows_v, sem):
        wid = lax.axis_index("s") * 2 + lax.axis_index("c")
        base = wid * rows_w
        pltpu.sync_copy(tok_hbm.at[pl.ds(base, rows_w)], idx_v)
        pltpu.async_copy(st_hbm.at[idx_v], srows_v, sem).wait()
        pltpu.sync_copy(srows_v, xs_out.at[pl.ds(base, rows_w)])

        def hsub(c, carry):
            b = base + c * 32
            pltpu.sync_copy(tok_hbm.at[pl.ds(b, 32)], idx32_v)
            pltpu.async_copy(h_hbm.at[idx32_v], rows_v, sem).wait()
            pltpu.sync_copy(rows_v, xh_out.at[pl.ds(b, 32)])
            return carry
        lax.fori_loop(0, rows_w // 32, hsub, 0)

    return g1(h, stage, tok_sorted)


def _mlp_body(tr, er, lr, hr, xh, xf, ws, w1h, w1f, b1, w2, b2, w3, b3, out):
    j = pl.program_id(0)
    t = tr[j]
    lo = lr[j]
    hi = hr[j]
    rows = t * _TM + lax.broadcasted_iota(jnp.int32, (_TM, 1), 0)
    vmask = (rows >= lo) & (rows < hi)
    z1 = (jnp.dot(xh[...], w1h[0], preferred_element_type=jnp.float32)
          + jnp.dot(xf[...], w1f[0], preferred_element_type=jnp.float32)
          + b1[0])
    h1 = _gelu(z1)
    h2 = _gelu(jnp.dot(h1, w2[0], preferred_element_type=jnp.float32) + b2[0])
    y = jnp.dot(h2, w3[0], preferred_element_type=jnp.float32) + b3[0]
    y = y * ws[0, 0, :][:, None]
    contrib = jnp.where(vmask, y, 0.0)
    prev_t = tr[jnp.maximum(j - 1, 0)]
    first = (j == 0) | (t != prev_t)

    @pl.when(first)
    def _():
        out[...] = contrib

    @pl.when(jnp.logical_not(first))
    def _():
        out[...] += contrib


def _mlp_call(meta, xh, xf, ws3, We1h, We1f, be1, We2, be2, We3, be3):
    wu_tile, wu_expert, wu_lo, wu_hi = meta
    grid_spec = pltpu.PrefetchScalarGridSpec(
        num_scalar_prefetch=4,
        grid=(_NW,),
        in_specs=[
            pl.BlockSpec((_TM, _D), lambda j, tr, er, lr, hr: (tr[j], 0)),
            pl.BlockSpec((_TM, _NC * _FB), lambda j, tr, er, lr, hr: (tr[j], 0)),
            pl.BlockSpec((1, 1, _TM), lambda j, tr, er, lr, hr: (tr[j], 0, 0)),
            pl.BlockSpec((1, _D, _H),
                         lambda j, tr, er, lr, hr: (er[j], 0, 0)),
            pl.BlockSpec((1, _NC * _FB, _H),
                         lambda j, tr, er, lr, hr: (er[j], 0, 0)),
            pl.BlockSpec((1, 1, _H), lambda j, tr, er, lr, hr: (er[j], 0, 0)),
            pl.BlockSpec((1, _H, _H), lambda j, tr, er, lr, hr: (er[j], 0, 0)),
            pl.BlockSpec((1, 1, _H), lambda j, tr, er, lr, hr: (er[j], 0, 0)),
            pl.BlockSpec((1, _H, _D), lambda j, tr, er, lr, hr: (er[j], 0, 0)),
            pl.BlockSpec((1, 1, _D), lambda j, tr, er, lr, hr: (er[j], 0, 0)),
        ],
        out_specs=pl.BlockSpec((_TM, _D), lambda j, tr, er, lr, hr: (tr[j], 0)),
    )
    return pl.pallas_call(
        _mlp_body,
        grid_spec=grid_spec,
        out_shape=jax.ShapeDtypeStruct((_KP, _D), jnp.float32),
        compiler_params=pltpu.CompilerParams(
            dimension_semantics=("arbitrary",),
            vmem_limit_bytes=100 * 1024 * 1024,
        ),
    )(wu_tile, wu_expert, wu_lo, wu_hi,
      xh, xf, ws3, We1h, We1f, be1, We2, be2, We3, be3)


def _sc_combine(hidden, ys, pos):
    mesh = plsc.VectorSubcoreMesh(core_axis_name="c", subcore_axis_name="s")
    tok_w = _B // _NWORK
    st = 16
    lanes = _D // 16

    @functools.partial(
        pl.kernel, mesh=mesh,
        out_type=jax.ShapeDtypeStruct((_B, _D), jnp.float32),
        scratch_types=[pltpu.VMEM((2 * st,), jnp.int32),
                       pltpu.VMEM((2 * st, _D), jnp.float32),
                       pltpu.VMEM((st, _D), jnp.float32),
                       pltpu.SemaphoreType.DMA],
    )
    def g2(hid_hbm, ys_hbm, pos_hbm, out_hbm, idx_v, ys_v, hid_v, sem):
        wid = lax.axis_index("s") * 2 + lax.axis_index("c")
        tbase = wid * tok_w

        def sub(c, carry):
            t0 = tbase + c * st
            pltpu.sync_copy(pos_hbm.at[pl.ds(2 * t0, 2 * st)], idx_v)
            cp = pltpu.async_copy(ys_hbm.at[idx_v], ys_v, sem)
            pltpu.sync_copy(hid_hbm.at[pl.ds(t0, st)], hid_v)
            cp.wait()

            def row(r, c2):
                def lane(l, c3):
                    o = l * 16
                    acc = (hid_v[r, pl.ds(o, 16)]
                           + ys_v[2 * r, pl.ds(o, 16)]
                           + ys_v[2 * r + 1, pl.ds(o, 16)])
                    hid_v[r, pl.ds(o, 16)] = acc
                    return c3
                return lax.fori_loop(0, lanes, lane, c2)
            lax.fori_loop(0, st, row, 0)
            pltpu.sync_copy(hid_v, out_hbm.at[pl.ds(t0, st)])
            return carry
        lax.fori_loop(0, tok_w // st, sub, 0)

    return g2(hidden, ys, pos)


def kernel(hidden, feature_bank, expert_bank_idx, ln_gamma, ln_beta,
           rW1, rb1, rW2, rb2, We1, be1, We2, be2, We3, be3, alpha):
    f32 = jnp.float32
    stage = feature_bank.reshape(_B, _NC * _FB)
    g2d = ln_gamma.reshape(1, _D)
    b2d = ln_beta.reshape(1, _D)
    rb1_2d = rb1.reshape(1, _RH)
    rW2p = jnp.zeros((_RH, _EPAD), f32).at[:, :_E].set(rW2)
    rb2p = jnp.full((_EPAD,), -1e30, f32).at[:_E].set(rb2).reshape(1, _EPAD)

    h_n, idx, wp = _router_call(hidden, stage, g2d, b2d, rW1,
                                rb1_2d, rW2p, rb2p)

    (pos, perm, tok_sorted, e_sorted,
     wu_tile, wu_expert, wu_lo, wu_hi) = _dispatch_meta(idx)

    ws3 = (alpha * wp.reshape(-1)[perm]).reshape(_NT, 1, _TM)

    xh_sorted, xs_sorted = _sc_gather(h_n, stage, tok_sorted)

    Wf = We1[:, _D:, :].reshape(_E, _FPE, _FB * _H)
    M = (expert_bank_idx[..., None]
         == jnp.arange(_NC, dtype=jnp.int32)).astype(f32)
    We1fx = jnp.einsum('epc,epk->eck', M, Wf).reshape(_E, _NC * _FB, _H)
    ys = _mlp_call((wu_tile, wu_expert, wu_lo, wu_hi),
                   xh_sorted, xs_sorted, ws3,
                   We1, We1fx, be1.reshape(_E, 1, _H), We2,
                   be2.reshape(_E, 1, _H), We3, be3.reshape(_E, 1, _D))

    return _sc_combine(hidden, ys, pos)


# speedup vs baseline: 1.9740x; 1.0531x over previous
import functools

import jax
import jax.numpy as jnp
from jax import lax
from jax.experimental import pallas as pl
from jax.experimental.pallas import tpu as pltpu
from jax.experimental.pallas import tpu_sc as plsc

_B = 2048
_D = 2048
_E = 8
_NC = 16
_FB = 16
_FPE = 2
_H = 1024
_RH = 1024
_K = 2
_EPAD = 128

_TMR = 256
_TM = 256
_KP = _B * _K
_NT = _KP // _TM
_NW = _NT + _E - 1
_NWORK = 32


def _gelu(x):
    return 0.5 * x * (1.0 + lax.erf(x * 0.7071067811865476))


def _router_body(hid, st, g, b, w1h, w1f, b1, w2, b2, h_out, idx_out, wp_out):
    x = hid[...]
    mu = jnp.mean(x, axis=-1, keepdims=True)
    xc = x - mu
    var = jnp.mean(xc * xc, axis=-1, keepdims=True)
    hn = xc * lax.rsqrt(var + 1e-5) * g[...] + b[...]
    h_out[...] = hn
    z = (jnp.dot(hn, w1h[...], preferred_element_type=jnp.float32)
         + jnp.dot(st[...], w1f[...], preferred_element_type=jnp.float32)
         + b1[...])
    z = _gelu(z)
    logits = jnp.dot(z, w2[...], preferred_element_type=jnp.float32) + b2[...]
    cols = lax.broadcasted_iota(jnp.int32, logits.shape, 1)
    big = jnp.int32(2**30)
    m1 = jnp.max(logits, axis=-1, keepdims=True)
    i1 = jnp.min(jnp.where(logits == m1, cols, big), axis=-1, keepdims=True)
    l2 = jnp.where(cols == i1, -jnp.inf, logits)
    m2 = jnp.max(l2, axis=-1, keepdims=True)
    i2 = jnp.min(jnp.where(l2 == m2, cols, big), axis=-1, keepdims=True)
    w1v = 1.0 / (1.0 + jnp.exp(m2 - m1))
    w2v = 1.0 / (1.0 + jnp.exp(m1 - m2))
    idx_out[...] = jnp.concatenate([i1, i2], axis=-1)
    wp_out[...] = jnp.concatenate([w1v, w2v], axis=-1)


def _router_call(hidden, stage, g2d, b2d, rW1, rb1_2d, rW2p, rb2p):
    return pl.pallas_call(
        _router_body,
        grid=(_B // _TMR,),
        in_specs=[
            pl.BlockSpec((_TMR, _D), lambda i: (i, 0)),
            pl.BlockSpec((_TMR, _NC * _FB), lambda i: (i, 0)),
            pl.BlockSpec((1, _D), lambda i: (0, 0)),
            pl.BlockSpec((1, _D), lambda i: (0, 0)),
            pl.BlockSpec((_D, _RH), lambda i: (0, 0)),
            pl.BlockSpec((_NC * _FB, _RH),
                         lambda i: (_D // (_NC * _FB), 0)),
            pl.BlockSpec((1, _RH), lambda i: (0, 0)),
            pl.BlockSpec((_RH, _EPAD), lambda i: (0, 0)),
            pl.BlockSpec((1, _EPAD), lambda i: (0, 0)),
        ],
        out_specs=[
            pl.BlockSpec((_TMR, _D), lambda i: (i, 0)),
            pl.BlockSpec((_TMR, _K), lambda i: (i, 0)),
            pl.BlockSpec((_TMR, _K), lambda i: (i, 0)),
        ],
        out_shape=[
            jax.ShapeDtypeStruct((_B, _D), jnp.float32),
            jax.ShapeDtypeStruct((_B, _K), jnp.int32),
            jax.ShapeDtypeStruct((_B, _K), jnp.float32),
        ],
    )(hidden, stage, g2d, b2d, rW1, rW1, rb1_2d, rW2p, rb2p)


def _dispatch_meta(idx):
    idx_flat = idx.reshape(-1).astype(jnp.int32)
    onehot = (idx_flat[:, None] == jnp.arange(_E, dtype=jnp.int32)[None, :])
    csum = jnp.cumsum(onehot.astype(jnp.int32), axis=0)
    counts = csum[-1]
    rank = jnp.take_along_axis(csum, idx_flat[:, None], axis=1)[:, 0] - 1
    offsets = jnp.concatenate(
        [jnp.zeros((1,), jnp.int32), jnp.cumsum(counts)]).astype(jnp.int32)
    pos = offsets[idx_flat] + rank
    perm = jnp.zeros((_KP,), jnp.int32).at[pos].set(
        jnp.arange(_KP, dtype=jnp.int32))
    tok_sorted = perm // _K
    e_sorted = idx_flat[perm]

    starts, ends = offsets[:-1], offsets[1:]
    first_tile = starts // _TM
    last_tile = jnp.where(counts > 0, (ends - 1) // _TM, first_tile)
    ntiles = jnp.where(counts > 0, last_tile - first_tile + 1, 0)
    wu_end = jnp.cumsum(ntiles)
    wu_start = wu_end - ntiles
    total = wu_end[-1]
    j = jnp.arange(_NW, dtype=jnp.int32)
    e_j = jnp.sum((j[:, None] >= wu_end[None, :]).astype(jnp.int32), axis=1)
    valid = j < total
    e_jc = jnp.minimum(e_j, _E - 1)
    tile_j = first_tile[e_jc] + (j - wu_start[e_jc])
    last_e = jnp.max(jnp.where(ntiles > 0, jnp.arange(_E), -1)).astype(jnp.int32)
    wu_tile = jnp.where(valid, tile_j, _NT - 1).astype(jnp.int32)
    wu_expert = jnp.where(valid, e_jc, last_e).astype(jnp.int32)
    wu_lo = jnp.where(valid, starts[wu_expert], 0).astype(jnp.int32)
    wu_hi = jnp.where(valid, ends[wu_expert], 0).astype(jnp.int32)
    return pos, perm, tok_sorted, e_sorted, wu_tile, wu_expert, wu_lo, wu_hi


def _sc_gather(h, stage, tok_sorted):
    mesh = plsc.VectorSubcoreMesh(core_axis_name="c", subcore_axis_name="s")
    rows_w = _KP // _NWORK
    sf = _NC * _FB
    ch = 16
    nch = rows_w // ch

    @functools.partial(
        pl.kernel, mesh=mesh,
        out_type=[jax.ShapeDtypeStruct((_KP, _D), jnp.float32),
                  jax.ShapeDtypeStruct((_KP, sf), jnp.float32)],
        scratch_types=[pltpu.VMEM((rows_w,), jnp.int32),
                       pltpu.VMEM((ch, _D), jnp.float32),
                       pltpu.VMEM((ch, _D), jnp.float32),
                       pltpu.VMEM((rows_w, sf), jnp.float32),
                       pltpu.SemaphoreType.DMA,
                       pltpu.SemaphoreType.DMA,
                       pltpu.SemaphoreType.DMA],
    )
    def g1(h_hbm, st_hbm, tok_hbm, xh_out, xs_out,
           idx_v, rows_a, rows_b, srows_v, sem_a, sem_b, sem_s):
        wid = lax.axis_index("s") * 2 + lax.axis_index("c")
        base = wid * rows_w
        pltpu.sync_copy(tok_hbm.at[pl.ds(base, rows_w)], idx_v)
        cs = pltpu.async_copy(st_hbm.at[idx_v], srows_v, sem_s)
        pltpu.async_copy(h_hbm.at[idx_v.at[pl.ds(0, ch)]], rows_a, sem_a)
        pltpu.async_copy(h_hbm.at[idx_v.at[pl.ds(ch, ch)]], rows_b, sem_b)

        def step(c, buf, sem):
            pltpu.make_async_copy(h_hbm.at[pl.ds(0, ch)], buf, sem).wait()
            pltpu.sync_copy(buf, xh_out.at[pl.ds(base + c * ch, ch)])

            @pl.when(c + 2 < nch)
            def _():
                pltpu.async_copy(
                    h_hbm.at[idx_v.at[pl.ds((c + 2) * ch, ch)]], buf, sem)

        def pair(c2, carry):
            step(2 * c2, rows_a, sem_a)
            step(2 * c2 + 1, rows_b, sem_b)
            return carry
        lax.fori_loop(0, nch // 2, pair, 0)
        cs.wait()
        pltpu.sync_copy(srows_v, xs_out.at[pl.ds(base, rows_w)])

    return g1(h, stage, tok_sorted)


def _mlp_body(tr, er, lr, hr, xh, xf, ws, w1h, w1f, b1, w2, b2, w3, b3, out):
    j = pl.program_id(0)
    t = tr[j]
    lo = lr[j]
    hi = hr[j]
    rows = t * _TM + lax.broadcasted_iota(jnp.int32, (_TM, 1), 0)
    vmask = (rows >= lo) & (rows < hi)
    z1 = (jnp.dot(xh[...], w1h[0], preferred_element_type=jnp.float32)
          + jnp.dot(xf[...], w1f[0], preferred_element_type=jnp.float32)
          + b1[0])
    h1 = _gelu(z1)
    h2 = _gelu(jnp.dot(h1, w2[0], preferred_element_type=jnp.float32) + b2[0])
    y = jnp.dot(h2, w3[0], preferred_element_type=jnp.float32) + b3[0]
    y = y * ws[0, 0, :][:, None]
    contrib = jnp.where(vmask, y, 0.0)
    prev_t = tr[jnp.maximum(j - 1, 0)]
    first = (j == 0) | (t != prev_t)

    @pl.when(first)
    def _():
        out[...] = contrib

    @pl.when(jnp.logical_not(first))
    def _():
        out[...] += contrib


def _mlp_call(meta, xh, xf, ws3, We1h, We1f, be1, We2, be2, We3, be3):
    wu_tile, wu_expert, wu_lo, wu_hi = meta
    grid_spec = pltpu.PrefetchScalarGridSpec(
        num_scalar_prefetch=4,
        grid=(_NW,),
        in_specs=[
            pl.BlockSpec((_TM, _D), lambda j, tr, er, lr, hr: (tr[j], 0)),
            pl.BlockSpec((_TM, _NC * _FB), lambda j, tr, er, lr, hr: (tr[j], 0)),
            pl.BlockSpec((1, 1, _TM), lambda j, tr, er, lr, hr: (tr[j], 0, 0)),
            pl.BlockSpec((1, _D, _H),
                         lambda j, tr, er, lr, hr: (er[j], 0, 0)),
            pl.BlockSpec((1, _NC * _FB, _H),
                         lambda j, tr, er, lr, hr: (er[j], 0, 0)),
            pl.BlockSpec((1, 1, _H), lambda j, tr, er, lr, hr: (er[j], 0, 0)),
            pl.BlockSpec((1, _H, _H), lambda j, tr, er, lr, hr: (er[j], 0, 0)),
            pl.BlockSpec((1, 1, _H), lambda j, tr, er, lr, hr: (er[j], 0, 0)),
            pl.BlockSpec((1, _H, _D), lambda j, tr, er, lr, hr: (er[j], 0, 0)),
            pl.BlockSpec((1, 1, _D), lambda j, tr, er, lr, hr: (er[j], 0, 0)),
        ],
        out_specs=pl.BlockSpec((_TM, _D), lambda j, tr, er, lr, hr: (tr[j], 0)),
    )
    return pl.pallas_call(
        _mlp_body,
        grid_spec=grid_spec,
        out_shape=jax.ShapeDtypeStruct((_KP, _D), jnp.float32),
        compiler_params=pltpu.CompilerParams(
            dimension_semantics=("arbitrary",),
            vmem_limit_bytes=100 * 1024 * 1024,
        ),
    )(wu_tile, wu_expert, wu_lo, wu_hi,
      xh, xf, ws3, We1h, We1f, be1, We2, be2, We3, be3)


def _sc_combine(hidden, ys, pos):
    mesh = plsc.VectorSubcoreMesh(core_axis_name="c", subcore_axis_name="s")
    tok_w = _B // _NWORK
    st = 8
    nsub = tok_w // st
    unroll = 4
    gpi = _D // (16 * unroll)

    @functools.partial(
        pl.kernel, mesh=mesh,
        out_type=jax.ShapeDtypeStruct((_B, _D), jnp.float32),
        scratch_types=[pltpu.VMEM((2 * tok_w,), jnp.int32),
                       pltpu.VMEM((2 * st, _D), jnp.float32),
                       pltpu.VMEM((2 * st, _D), jnp.float32),
                       pltpu.VMEM((st, _D), jnp.float32),
                       pltpu.VMEM((st, _D), jnp.float32),
                       pltpu.SemaphoreType.DMA,
                       pltpu.SemaphoreType.DMA,
                       pltpu.SemaphoreType.DMA,
                       pltpu.SemaphoreType.DMA],
    )
    def g2(hid_hbm, ys_hbm, pos_hbm, out_hbm, idx_v, ys_a, ys_b,
           hid_a, hid_b, sem_ya, sem_yb, sem_ha, sem_hb):
        wid = lax.axis_index("s") * 2 + lax.axis_index("c")
        tbase = wid * tok_w
        pltpu.sync_copy(pos_hbm.at[pl.ds(2 * tbase, 2 * tok_w)], idx_v)

        def fire(c, ys_buf, hid_buf, sem_y, sem_h):
            pltpu.async_copy(
                ys_hbm.at[idx_v.at[pl.ds(2 * c * st, 2 * st)]], ys_buf, sem_y)
            pltpu.async_copy(
                hid_hbm.at[pl.ds(tbase + c * st, st)], hid_buf, sem_h)

        fire(0, ys_a, hid_a, sem_ya, sem_ha)
        fire(1, ys_b, hid_b, sem_yb, sem_hb)

        def step(c, ys_buf, hid_buf, sem_y, sem_h):
            pltpu.make_async_copy(ys_hbm.at[pl.ds(0, 2 * st)], ys_buf,
                                  sem_y).wait()
            pltpu.make_async_copy(hid_hbm.at[pl.ds(0, st)], hid_buf,
                                  sem_h).wait()

            def grp(k, carry):
                r = k // gpi
                ob = (k - r * gpi) * (16 * unroll)
                for u in range(unroll):
                    o = ob + u * 16
                    hid_buf[r, pl.ds(o, 16)] = (
                        hid_buf[r, pl.ds(o, 16)]
                        + ys_buf[2 * r, pl.ds(o, 16)]
                        + ys_buf[2 * r + 1, pl.ds(o, 16)])
                return carry
            lax.fori_loop(0, st * gpi, grp, 0)
            pltpu.sync_copy(hid_buf, out_hbm.at[pl.ds(tbase + c * st, st)])

            @pl.when(c + 2 < nsub)
            def _():
                fire(c + 2, ys_buf, hid_buf, sem_y, sem_h)

        def pair(c2, carry):
            step(2 * c2, ys_a, hid_a, sem_ya, sem_ha)
            step(2 * c2 + 1, ys_b, hid_b, sem_yb, sem_hb)
            return carry
        lax.fori_loop(0, nsub // 2, pair, 0)

    return g2(hidden, ys, pos)


def kernel(hidden, feature_bank, expert_bank_idx, ln_gamma, ln_beta,
           rW1, rb1, rW2, rb2, We1, be1, We2, be2, We3, be3, alpha):
    f32 = jnp.float32
    stage = feature_bank.reshape(_B, _NC * _FB)
    g2d = ln_gamma.reshape(1, _D)
    b2d = ln_beta.reshape(1, _D)
    rb1_2d = rb1.reshape(1, _RH)
    rW2p = jnp.zeros((_RH, _EPAD), f32).at[:, :_E].set(rW2)
    rb2p = jnp.full((_EPAD,), -1e30, f32).at[:_E].set(rb2).reshape(1, _EPAD)

    h_n, idx, wp = _router_call(hidden, stage, g2d, b2d, rW1,
                                rb1_2d, rW2p, rb2p)

    (pos, perm, tok_sorted, e_sorted,
     wu_tile, wu_expert, wu_lo, wu_hi) = _dispatch_meta(idx)

    ws3 = (alpha * wp.reshape(-1)[perm]).reshape(_NT, 1, _TM)

    xh_sorted, xs_sorted = _sc_gather(h_n, stage, tok_sorted)

    Wf = We1[:, _D:, :].reshape(_E, _FPE, _FB * _H)
    M = (expert_bank_idx[..., None]
         == jnp.arange(_NC, dtype=jnp.int32)).astype(f32)
    We1fx = jnp.einsum('epc,epk->eck', M, Wf).reshape(_E, _NC * _FB, _H)
    ys = _mlp_call((wu_tile, wu_expert, wu_lo, wu_hi),
                   xh_sorted, xs_sorted, ws3,
                   We1, We1fx, be1.reshape(_E, 1, _H), We2,
                   be2.reshape(_E, 1, _H), We3, be3.reshape(_E, 1, _D))

    return _sc_combine(hidden, ys, pos)
